# baseline (speedup 1.0000x reference)
import jax
import jax.numpy as jnp
from jax.experimental import pallas as pl

T = 4096
D = 2048
E = 8
K = 2
F = 512
FS = 512
RSF = 2.5


def _silu_and_mul(x):
    g, u = jnp.split(x, 2, axis=-1)
    return jax.nn.silu(g) * u


def kernel(hidden_states, gate_w, w_gate_up, w_down, shared_gate_up, shared_down):
    x = hidden_states.reshape(-1, D)
    router_logits = x @ gate_w
    scores = jax.nn.sigmoid(router_logits.astype(jnp.float32))
    topk_w, topk_idx = jax.lax.top_k(scores, K)
    topk_w = topk_w / jnp.sum(topk_w, axis=-1, keepdims=True)
    onehot = jax.nn.one_hot(topk_idx, E, dtype=x.dtype)
    combine = jnp.einsum('tk,tke->te', topk_w.astype(x.dtype), onehot)

    xb = x.astype(jnp.bfloat16)
    routed = jnp.zeros_like(x)
    for e in range(E):
        h = _silu_and_mul(jnp.dot(xb, w_gate_up[e].astype(jnp.bfloat16),
                                  preferred_element_type=jnp.float32))
        hd = jnp.dot(h.astype(jnp.bfloat16), w_down[e].astype(jnp.bfloat16),
                     preferred_element_type=jnp.float32)
        routed = routed + combine[:, e:e + 1] * hd
    shared = jnp.dot(
        _silu_and_mul(jnp.dot(xb, shared_gate_up.astype(jnp.bfloat16),
                              preferred_element_type=jnp.float32)).astype(jnp.bfloat16),
        shared_down.astype(jnp.bfloat16), preferred_element_type=jnp.float32)
    final = routed * RSF + shared
    return final.reshape(T, D)


# speedup vs baseline: 1.4719x; 1.4719x over previous
import functools

import jax
import jax.numpy as jnp
from jax.experimental import pallas as pl
from jax.experimental.pallas import tpu as pltpu

T = 4096
D = 2048
E = 8
K = 2
F = 512
FS = 512
RSF = 2.5

TB = 512
NB = T // TB
EPAD = 128


def _silu_and_mul(x):
    g = x[:, :F]
    u = x[:, F:]
    return (g * jax.nn.sigmoid(g)) * u


def _moe_body(xb_ref, gw_ref, wgu_ref, wd_ref, sgu_ref, sdn_ref, out_ref, coef_ref):
    e = pl.program_id(1)
    xb = xb_ref[...]

    @pl.when(e == 0)
    def _router():
        logits = jnp.dot(xb, gw_ref[...], preferred_element_type=jnp.float32)
        lane = jax.lax.broadcasted_iota(jnp.int32, (TB, EPAD), 1)
        neg = jnp.float32(-1e30)
        s = jnp.where(lane < E, jax.nn.sigmoid(logits), neg)
        m1 = jnp.max(s, axis=1, keepdims=True)
        i1 = jnp.argmax(s, axis=1)[:, None]
        s2 = jnp.where(lane == i1, neg, s)
        m2 = jnp.max(s2, axis=1, keepdims=True)
        i2 = jnp.argmax(s2, axis=1)[:, None]
        denom = m1 + m2
        coef_ref[...] = (jnp.where(lane == i1, m1 / denom, 0.0)
                         + jnp.where(lane == i2, m2 / denom, 0.0))

    prev = jnp.where(e == 0, jnp.zeros_like(out_ref), out_ref[...])

    @pl.when(e < E)
    def _routed():
        lane = jax.lax.broadcasted_iota(jnp.int32, (TB, EPAD), 1)
        c = jnp.sum(jnp.where(lane == e, coef_ref[...], 0.0), axis=1, keepdims=True)
        h = _silu_and_mul(jnp.dot(xb, wgu_ref[0], preferred_element_type=jnp.float32))
        hd = jnp.dot(h.astype(jnp.bfloat16), wd_ref[0],
                     preferred_element_type=jnp.float32)
        out_ref[...] = prev + (RSF * c) * hd

    @pl.when(e == E)
    def _shared():
        h = _silu_and_mul(jnp.dot(xb, sgu_ref[...], preferred_element_type=jnp.float32))
        hd = jnp.dot(h.astype(jnp.bfloat16), sdn_ref[...],
                     preferred_element_type=jnp.float32)
        out_ref[...] = prev + hd


@jax.jit
def kernel(hidden_states, gate_w, w_gate_up, w_down, shared_gate_up, shared_down):
    xb = hidden_states.astype(jnp.bfloat16)
    gw = jnp.pad(gate_w, ((0, 0), (0, EPAD - E))).astype(jnp.bfloat16)
    wgu = w_gate_up.astype(jnp.bfloat16)
    wd = w_down.astype(jnp.bfloat16)
    sgu = shared_gate_up.astype(jnp.bfloat16)
    sdn = shared_down.astype(jnp.bfloat16)

    grid = (NB, E + 1)
    out = pl.pallas_call(
        _moe_body,
        grid=grid,
        in_specs=[
            pl.BlockSpec((TB, D), lambda b, e: (b, 0)),
            pl.BlockSpec((D, EPAD), lambda b, e: (0, 0)),
            pl.BlockSpec((1, D, 2 * F), lambda b, e: (jnp.minimum(e, E - 1), 0, 0)),
            pl.BlockSpec((1, F, D), lambda b, e: (jnp.minimum(e, E - 1), 0, 0)),
            pl.BlockSpec((D, 2 * FS), lambda b, e: (0, 0)),
            pl.BlockSpec((FS, D), lambda b, e: (0, 0)),
        ],
        out_specs=pl.BlockSpec((TB, D), lambda b, e: (b, 0)),
        out_shape=jax.ShapeDtypeStruct((T, D), jnp.float32),
        scratch_shapes=[pltpu.VMEM((TB, EPAD), jnp.float32)],
        compiler_params=pltpu.CompilerParams(
            dimension_semantics=("parallel", "arbitrary"),
        ),
    )(xb, gw, wgu, wd, sgu, sdn)
    return out
